# Initial kernel scaffold; baseline (speedup 1.0000x reference)
import jax
import jax.numpy as jnp
from jax.experimental import pallas as pl


def kernel(x, perms):
    raise NotImplementedError("write your pallas kernel here")


# speedup vs baseline: 2.7950x; 2.7950x over previous
import functools

import jax
import jax.numpy as jnp
from jax import lax
from jax.experimental import pallas as pl
from jax.experimental.pallas import tpu as pltpu
from jax.experimental.pallas import tpu_sc as plsc

C = 64
NBINS = C + 1
ALPHA_LO = 0.5 - 0.1
ALPHA_HI = 0.5 + 0.1
NC = 2
NS = 16
NW = NC * NS
L = 16


def _sc_body(B, E, R, x_hbm, l_hbm, r_hbm, out_hbm, xt, lv, rv, hist, ot, sem):
    del sem
    rows_per_w = B // NW
    nblocks = rows_per_w // R

    cid = lax.axis_index("c")
    sid = lax.axis_index("s")
    wid = sid * NC + cid
    row0 = wid * rows_per_w

    lane = lax.iota(jnp.int32, L)
    lane_hist = lane * NBINS
    lane_x = lane * E
    lane_out = lane * C
    ones = jnp.ones((L,), jnp.int32)
    zeros = jnp.zeros((L,), jnp.int32)

    pltpu.sync_copy(l_hbm, lv)
    pltpu.sync_copy(r_hbm, rv)

    def zero_ot(i, _):
        ot[pl.ds(i * L, L)] = zeros
        return 0
    lax.fori_loop(0, (R * C) // L, zero_ot, 0)

    def per_block(bi, _):
        r0 = row0 + bi * R
        pltpu.sync_copy(x_hbm.at[pl.ds(r0 * E, R * E)], xt)

        def zero_hist(i, _):
            hist[pl.ds(i * L, L)] = zeros
            return 0
        lax.fori_loop(0, (L * NBINS) // L, zero_hist, 0)

        def per_elem(e, _):
            es = jnp.full((L,), e, jnp.int32)
            xv = plsc.load_gather(xt, [lane_x + es])
            lb = plsc.load_gather(lv, [es])
            rb = plsc.load_gather(rv, [es])
            below = xv <= ALPHA_LO
            above = xv >= ALPHA_HI
            sel = jnp.where(below, lb, jnp.where(above, rb, C))
            plsc.addupdate_scatter(hist, [lane_hist + sel], ones)
            return 0
        lax.fori_loop(0, E, per_elem, 0)

        def argmax_step(c, carry):
            m, am = carry
            cs = jnp.full((L,), c, jnp.int32)
            v = plsc.load_gather(hist, [lane_hist + cs])
            better = v > m
            return jnp.where(better, v, m), jnp.where(better, cs, am)
        m0 = jnp.full((L,), -1, jnp.int32)
        _, am = lax.fori_loop(0, C, argmax_step, (m0, zeros))

        plsc.store_scatter(ot, [lane_out + am], ones)
        pltpu.sync_copy(ot, out_hbm.at[pl.ds(r0 * C, R * C)])
        plsc.store_scatter(ot, [lane_out + am], zeros)
        return 0

    lax.fori_loop(0, nblocks, per_block, 0)


def kernel(x, perms):
    B, E = x.shape
    left = perms[:, 0].astype(jnp.int32)
    right = perms[:, 1].astype(jnp.int32)
    R = 16

    mesh = plsc.VectorSubcoreMesh(
        core_axis_name="c", subcore_axis_name="s",
        num_cores=NC, num_subcores=NS)

    run = pl.kernel(
        functools.partial(_sc_body, B, E, R),
        out_type=jax.ShapeDtypeStruct((B * C,), jnp.int32),
        mesh=mesh,
        compiler_params=pltpu.CompilerParams(needs_layout_passes=False),
        scratch_types=[
            pltpu.VMEM((R * E,), jnp.float32),
            pltpu.VMEM((E,), jnp.int32),
            pltpu.VMEM((E,), jnp.int32),
            pltpu.VMEM((L * NBINS,), jnp.int32),
            pltpu.VMEM((R * C,), jnp.int32),
            pltpu.SemaphoreType.DMA,
        ],
    )
    out = run(x.reshape(B * E), left, right)
    return out.reshape(B, C).astype(jnp.int64)


# speedup vs baseline: 6.3273x; 2.2638x over previous
import functools

import jax
import jax.numpy as jnp
from jax import lax
from jax.experimental import pallas as pl
from jax.experimental.pallas import tpu as pltpu
from jax.experimental.pallas import tpu_sc as plsc

C = 64
NBINS = C + 1
ALPHA_LO = 0.5 - 0.1
ALPHA_HI = 0.5 + 0.1
NC = 2
NS = 16
NW = NC * NS
L = 16


def _sc_body(B, E, R, x_hbm, l_hbm, r_hbm, out_hbm,
             xt0, xt1, lv, rv, hist, ot, sem0, sem1):
    rows_per_w = B // NW
    nblocks = rows_per_w // R
    nchunks = E // L

    cid = lax.axis_index("c")
    sid = lax.axis_index("s")
    wid = sid * NC + cid
    row0 = wid * rows_per_w

    lane = lax.iota(jnp.int32, L)
    lane_x = lane * E
    lane_out = lane * C
    ones = jnp.ones((L,), jnp.int32)
    zeros = jnp.zeros((L,), jnp.int32)
    center16 = jnp.full((L,), C * L, jnp.int32)

    pltpu.sync_copy(l_hbm, lv)
    pltpu.sync_copy(r_hbm, rv)

    for i in range((R * C) // L):
        ot[pl.ds(i * L, L)] = zeros

    def x_rows(b):
        return x_hbm.at[pl.ds((row0 + b * R) * E, R * E)]

    def compute_block(bi, xt):
        for i in range((L * NBINS) // L):
            hist[pl.ds(i * L, L)] = zeros

        def per_chunk(ch):
            ch16 = ch * L
            lv16 = lv[pl.ds(ch16, L)]
            rv16 = rv[pl.ds(ch16, L)]
            bidx = lane_x + ch16
            for u in range(L):
                us = jnp.full((L,), u, jnp.int32)
                lb = jnp.take_along_axis(lv16, us, axis=0,
                                         mode="promise_in_bounds")
                rb = jnp.take_along_axis(rv16, us, axis=0,
                                         mode="promise_in_bounds")
                xv = plsc.load_gather(xt, [bidx + u])
                below = xv <= ALPHA_LO
                above = xv >= ALPHA_HI
                sel16 = jnp.where(below, lb, jnp.where(above, rb, center16))
                plsc.addupdate_scatter(hist, [sel16 + lane], ones)
        plsc.parallel_loop(0, nchunks)(per_chunk)

        m = jnp.full((L,), -1, jnp.int32)
        am = zeros
        for c in range(C):
            v = hist[pl.ds(c * L, L)]
            better = v > m
            m = jnp.where(better, v, m)
            am = jnp.where(better, jnp.full((L,), c, jnp.int32), am)

        plsc.store_scatter(ot, [lane_out + am], ones)
        pltpu.sync_copy(ot, out_hbm.at[pl.ds((row0 + bi * R) * C, R * C)])
        plsc.store_scatter(ot, [lane_out + am], zeros)

    pltpu.async_copy(x_rows(0), xt0, sem0)

    def per_pair(g, _):
        b0 = g * 2
        pltpu.async_copy(x_rows(b0 + 1), xt1, sem1)
        pltpu.make_async_copy(x_rows(b0), xt0, sem0).wait()
        compute_block(b0, xt0)
        nxt = jnp.minimum(b0 + 2, nblocks - 1)
        pltpu.async_copy(x_rows(nxt), xt0, sem0)
        pltpu.make_async_copy(x_rows(b0 + 1), xt1, sem1).wait()
        compute_block(b0 + 1, xt1)
        return 0

    lax.fori_loop(0, nblocks // 2, per_pair, 0)
    pltpu.make_async_copy(x_rows(0), xt0, sem0).wait()


def kernel(x, perms):
    B, E = x.shape
    left = perms[:, 0].astype(jnp.int32) * L
    right = perms[:, 1].astype(jnp.int32) * L
    R = 16

    mesh = plsc.VectorSubcoreMesh(
        core_axis_name="c", subcore_axis_name="s",
        num_cores=NC, num_subcores=NS)

    run = pl.kernel(
        functools.partial(_sc_body, B, E, R),
        out_type=jax.ShapeDtypeStruct((B * C,), jnp.int32),
        mesh=mesh,
        compiler_params=pltpu.CompilerParams(needs_layout_passes=False),
        scratch_types=[
            pltpu.VMEM((R * E,), jnp.float32),
            pltpu.VMEM((R * E,), jnp.float32),
            pltpu.VMEM((E,), jnp.int32),
            pltpu.VMEM((E,), jnp.int32),
            pltpu.VMEM((NBINS * L,), jnp.int32),
            pltpu.VMEM((R * C,), jnp.int32),
            pltpu.SemaphoreType.DMA,
            pltpu.SemaphoreType.DMA,
        ],
    )
    out = run(x.reshape(B * E), left, right)
    return out.reshape(B, C).astype(jnp.int64)


# speedup vs baseline: 11.2957x; 1.7852x over previous
import functools

import jax
import jax.numpy as jnp
from jax import lax
from jax.experimental import pallas as pl
from jax.experimental.pallas import tpu as pltpu
from jax.experimental.pallas import tpu_sc as plsc

C = 64
NBINS = C + 1
ALPHA_LO = 0.5 - 0.1
ALPHA_HI = 0.5 + 0.1
NC = 2
NS = 16
NW = NC * NS
L = 16


def _sc_body(B, E, R, x_hbm, lr_hbm, out_hbm,
             xt0, xt1, lrt, hist, ot, sem0, sem1):
    rows_per_w = B // NW
    nblocks = rows_per_w // R
    nchunks = E // L

    cid = lax.axis_index("c")
    sid = lax.axis_index("s")
    wid = sid * NC + cid
    row0 = wid * rows_per_w

    lane = lax.iota(jnp.int32, L)
    lane_rot = lane * (E + 1)
    lane_out = lane * C
    ones = jnp.ones((L,), jnp.int32)
    zeros = jnp.zeros((L,), jnp.int32)
    center16 = jnp.full((L,), C * L, jnp.int32)
    xwin = (L - 1) * (E + 1) + L

    pltpu.sync_copy(lr_hbm, lrt)

    for i in range((R * C) // L):
        ot[pl.ds(i * L, L)] = zeros

    def x_rows(b):
        return x_hbm.at[pl.ds((row0 + b * R) * E, R * E)]

    def hist_step(xv, lrp):
        rb = lax.shift_right_logical(lrp, 16)
        lb = lrp & 0xFFFF
        below = xv <= ALPHA_LO
        above = xv >= ALPHA_HI
        sel16 = jnp.where(below, lb, jnp.where(above, rb, center16))
        plsc.addupdate_scatter(hist, [sel16 + lane], ones)

    def compute_block(bi, xt):
        for i in range((L * NBINS) // L):
            hist[pl.ds(i * L, L)] = zeros

        def per_chunk(ch):
            ch16 = ch * L
            xs = xt.at[pl.ds(ch16, xwin)]
            lridx0 = lane + ch16
            for u in range(L):
                lrp = plsc.load_gather(lrt, [lridx0 + u])
                xv = plsc.load_gather(xs, [lane_rot + u])
                hist_step(xv, lrp)
        plsc.parallel_loop(0, nchunks - 1)(per_chunk)

        for u in range(L):
            e = E - L + u
            lrp = plsc.load_gather(lrt, [lane + e])
            wrap = (lane + e) >= E
            xidx = lane_rot + e - jnp.where(wrap, E, 0)
            xv = plsc.load_gather(xt, [xidx])
            hist_step(xv, lrp)

        m = jnp.full((L,), -1, jnp.int32)
        am = zeros
        for c in range(C):
            v = hist[pl.ds(c * L, L)]
            better = v > m
            m = jnp.where(better, v, m)
            am = jnp.where(better, jnp.full((L,), c * L, jnp.int32), am)
        am = lax.shift_right_logical(am, 4)

        plsc.store_scatter(ot, [lane_out + am], ones)
        pltpu.sync_copy(ot, out_hbm.at[pl.ds((row0 + bi * R) * C, R * C)])
        plsc.store_scatter(ot, [lane_out + am], zeros)

    pltpu.async_copy(x_rows(0), xt0, sem0)

    def per_pair(g, _):
        b0 = g * 2
        pltpu.async_copy(x_rows(b0 + 1), xt1, sem1)
        pltpu.make_async_copy(x_rows(b0), xt0, sem0).wait()
        compute_block(b0, xt0)
        nxt = jnp.minimum(b0 + 2, nblocks - 1)
        pltpu.async_copy(x_rows(nxt), xt0, sem0)
        pltpu.make_async_copy(x_rows(b0 + 1), xt1, sem1).wait()
        compute_block(b0 + 1, xt1)
        return 0

    lax.fori_loop(0, nblocks // 2, per_pair, 0)
    pltpu.make_async_copy(x_rows(0), xt0, sem0).wait()


def kernel(x, perms):
    B, E = x.shape
    left16 = perms[:, 0].astype(jnp.int32) * L
    right16 = perms[:, 1].astype(jnp.int32) * L
    lrp = left16 | (right16 << 16)
    lrp = jnp.concatenate([lrp, lrp[:L]])
    R = 16

    mesh = plsc.VectorSubcoreMesh(
        core_axis_name="c", subcore_axis_name="s",
        num_cores=NC, num_subcores=NS)

    run = pl.kernel(
        functools.partial(_sc_body, B, E, R),
        out_type=jax.ShapeDtypeStruct((B * C,), jnp.int32),
        mesh=mesh,
        compiler_params=pltpu.CompilerParams(needs_layout_passes=False),
        scratch_types=[
            pltpu.VMEM((R * E,), jnp.float32),
            pltpu.VMEM((R * E,), jnp.float32),
            pltpu.VMEM((E + L,), jnp.int32),
            pltpu.VMEM((NBINS * L,), jnp.int32),
            pltpu.VMEM((R * C,), jnp.int32),
            pltpu.SemaphoreType.DMA,
            pltpu.SemaphoreType.DMA,
        ],
    )
    out = run(x.reshape(B * E), lrp)
    return out.reshape(B, C).astype(jnp.int64)
